# baseline (speedup 1.0000x reference)
import functools

import jax
import jax.numpy as jnp
from jax import lax
from jax.experimental import pallas as pl
from jax.experimental.pallas import tpu as pltpu
from jax.experimental.pallas import tpu_sc as plsc

N_VOCAB = 100000
N_EMB = 64
CTX = 20
BATCH = 1024


@functools.cache
def _make_embed_sum():
    nc, ns, lanes = 2, 16, 16
    nw = nc * ns
    b_per_w = BATCH // nw
    rows_per_w = b_per_w * CTX
    chunk = 128
    n_chunks = rows_per_w // chunk

    mesh = plsc.VectorSubcoreMesh(core_axis_name="c", subcore_axis_name="s")

    pair_w = 2 * N_EMB

    @functools.partial(
        pl.kernel,
        mesh=mesh,
        out_type=jax.ShapeDtypeStruct((BATCH, N_EMB), jnp.float32),
        scratch_types=[
            pltpu.VMEM((rows_per_w,), jnp.int32),
            pltpu.VMEM((rows_per_w,), jnp.int32),
            pltpu.VMEM((rows_per_w, pair_w), jnp.float32),
            pltpu.VMEM((b_per_w, N_EMB), jnp.float32),
            pltpu.SemaphoreType.DMA,
        ],
    )
    def embed_sum(tok_hbm, table_hbm, out_hbm, idx_v, half_v, rows_v, acc_v, sem):
        wid = lax.axis_index("s") * nc + lax.axis_index("c")
        base = wid * rows_per_w
        pltpu.sync_copy(tok_hbm.at[pl.ds(base, rows_per_w)], idx_v)
        for g in range(rows_per_w // lanes):
            half_v[pl.ds(g * lanes, lanes)] = (
                idx_v[pl.ds(g * lanes, lanes)] >> 1
            )
        copies = []
        for j in range(n_chunks):
            copies.append(
                pltpu.async_copy(
                    table_hbm.at[half_v.at[pl.ds(j * chunk, chunk)]],
                    rows_v.at[pl.ds(j * chunk, chunk)],
                    sem,
                )
            )
        for c in copies:
            c.wait()

        def body(r, carry):
            accs = [jnp.zeros((lanes,), jnp.float32) for _ in range(N_EMB // lanes)]
            offs_a = (idx_v[pl.ds(r * CTX, lanes)] & 1) * N_EMB
            offs_b = (idx_v[pl.ds(r * CTX + CTX - lanes, lanes)] & 1) * N_EMB
            for j in range(CTX):
                trow = r * CTX + j
                off = offs_a[j] if j < lanes else offs_b[j - (CTX - lanes)]
                for c in range(N_EMB // lanes):
                    accs[c] = accs[c] + rows_v[trow, pl.ds(off + c * lanes, lanes)]
            for c in range(N_EMB // lanes):
                acc_v[r, pl.ds(c * lanes, lanes)] = accs[c]
            return carry

        lax.fori_loop(0, b_per_w, body, 0)
        pltpu.sync_copy(acc_v, out_hbm.at[pl.ds(wid * b_per_w, b_per_w)])

    return embed_sum


VB = 2048
MB = 256


def _mm_body(emb_ref, w_ref, b_ref, out_ref):
    out_ref[...] = (
        lax.dot_general(
            emb_ref[...].astype(jnp.bfloat16),
            w_ref[...].astype(jnp.bfloat16),
            (((1,), (1,)), ((), ())),
            preferred_element_type=jnp.float32,
        )
        + b_ref[...]
    )


def _projection(emb, fc_weight, bias2d):
    grid = (pl.cdiv(N_VOCAB, VB), BATCH // MB)
    return pl.pallas_call(
        _mm_body,
        grid=grid,
        in_specs=[
            pl.BlockSpec((MB, N_EMB), lambda i, j: (j, 0)),
            pl.BlockSpec((VB, N_EMB), lambda i, j: (i, 0)),
            pl.BlockSpec((1, VB), lambda i, j: (0, i)),
        ],
        out_specs=pl.BlockSpec((MB, VB), lambda i, j: (j, i)),
        out_shape=jax.ShapeDtypeStruct((BATCH, N_VOCAB), jnp.float32),
    )(emb, fc_weight, bias2d)


@jax.jit
def kernel(input_token, embedding_table, fc_weight, fc_bias):
    tok = input_token.reshape(-1).astype(jnp.int32)
    table_view = embedding_table.reshape(N_VOCAB // 2, 2 * N_EMB)
    emb = _make_embed_sum()(tok, table_view)
    return _projection(emb, fc_weight, fc_bias.reshape(1, -1))


# speedup vs baseline: 1.2224x; 1.2224x over previous
import functools

import jax
import jax.numpy as jnp
from jax import lax
from jax.experimental import pallas as pl
from jax.experimental.pallas import tpu as pltpu
from jax.experimental.pallas import tpu_sc as plsc

N_VOCAB = 100000
N_EMB = 64
CTX = 20
BATCH = 1024


@functools.cache
def _make_embed_sum():
    nc, ns, lanes = 2, 16, 16
    nw = nc * ns
    b_per_w = BATCH // nw
    rows_per_w = b_per_w * CTX
    chunk = 128
    n_chunks = rows_per_w // chunk

    mesh = plsc.VectorSubcoreMesh(core_axis_name="c", subcore_axis_name="s")

    pair_w = 2 * N_EMB

    @functools.partial(
        pl.kernel,
        mesh=mesh,
        out_type=jax.ShapeDtypeStruct((BATCH, N_EMB), jnp.float32),
        scratch_types=[
            pltpu.VMEM((rows_per_w,), jnp.int32),
            pltpu.VMEM((rows_per_w,), jnp.int32),
            pltpu.VMEM((rows_per_w, pair_w), jnp.float32),
            pltpu.VMEM((b_per_w, N_EMB), jnp.float32),
            pltpu.SemaphoreType.DMA,
        ],
    )
    def embed_sum(tok_hbm, table_hbm, out_hbm, idx_v, half_v, rows_v, acc_v, sem):
        wid = lax.axis_index("s") * nc + lax.axis_index("c")
        base = wid * rows_per_w
        pltpu.sync_copy(tok_hbm.at[pl.ds(base, rows_per_w)], idx_v)
        for g in range(rows_per_w // lanes):
            half_v[pl.ds(g * lanes, lanes)] = (
                idx_v[pl.ds(g * lanes, lanes)] >> 1
            )
        copies = []
        for j in range(n_chunks):
            copies.append(
                pltpu.async_copy(
                    table_hbm.at[half_v.at[pl.ds(j * chunk, chunk)]],
                    rows_v.at[pl.ds(j * chunk, chunk)],
                    sem,
                )
            )
        for c in copies:
            c.wait()

        def body(r, carry):
            accs = [jnp.zeros((lanes,), jnp.float32) for _ in range(N_EMB // lanes)]
            offs_a = (idx_v[pl.ds(r * CTX, lanes)] & 1) * N_EMB
            offs_b = (idx_v[pl.ds(r * CTX + CTX - lanes, lanes)] & 1) * N_EMB
            for j in range(CTX):
                trow = r * CTX + j
                off = offs_a[j] if j < lanes else offs_b[j - (CTX - lanes)]
                for c in range(N_EMB // lanes):
                    accs[c] = accs[c] + rows_v[trow, pl.ds(off + c * lanes, lanes)]
            for c in range(N_EMB // lanes):
                acc_v[r, pl.ds(c * lanes, lanes)] = accs[c]
            return carry

        lax.fori_loop(0, b_per_w, body, 0)
        pltpu.sync_copy(acc_v, out_hbm.at[pl.ds(wid * b_per_w, b_per_w)])

    return embed_sum


MB = 32


def _mm_body(emb_ref, wt_ref, b_ref, out_ref):
    out_ref[...] = (
        lax.dot_general(
            emb_ref[...].astype(jnp.bfloat16),
            wt_ref[...],
            (((1,), (0,)), ((), ())),
            preferred_element_type=jnp.float32,
        )
        + b_ref[...]
    )


def _projection(emb, wt_bf16, bias2d):
    grid = (BATCH // MB,)
    return pl.pallas_call(
        _mm_body,
        grid=grid,
        in_specs=[
            pl.BlockSpec((MB, N_EMB), lambda i: (i, 0)),
            pl.BlockSpec((N_EMB, N_VOCAB), lambda i: (0, 0)),
            pl.BlockSpec((1, N_VOCAB), lambda i: (0, 0)),
        ],
        out_specs=pl.BlockSpec((MB, N_VOCAB), lambda i: (i, 0)),
        out_shape=jax.ShapeDtypeStruct((BATCH, N_VOCAB), jnp.float32),
    )(emb, wt_bf16, bias2d)


@jax.jit
def kernel(input_token, embedding_table, fc_weight, fc_bias):
    tok = input_token.reshape(-1).astype(jnp.int32)
    table_view = embedding_table.reshape(N_VOCAB // 2, 2 * N_EMB)
    emb = _make_embed_sum()(tok, table_view)
    wt_bf16 = fc_weight.T.astype(jnp.bfloat16)
    return _projection(emb, wt_bf16, fc_bias.reshape(1, -1))


# speedup vs baseline: 1.2255x; 1.0025x over previous
import functools

import jax
import jax.numpy as jnp
from jax import lax
from jax.experimental import pallas as pl
from jax.experimental.pallas import tpu as pltpu
from jax.experimental.pallas import tpu_sc as plsc

N_VOCAB = 100000
N_EMB = 64
CTX = 20
BATCH = 1024


@functools.cache
def _make_embed_sum():
    nc, ns, lanes = 2, 16, 16
    nw = nc * ns
    b_per_w = BATCH // nw
    rows_per_w = b_per_w * CTX
    chunk = 128
    n_chunks = rows_per_w // chunk

    mesh = plsc.VectorSubcoreMesh(core_axis_name="c", subcore_axis_name="s")

    pair_w = 2 * N_EMB

    @functools.partial(
        pl.kernel,
        mesh=mesh,
        out_type=jax.ShapeDtypeStruct((BATCH, N_EMB), jnp.float32),
        scratch_types=[
            pltpu.VMEM((rows_per_w,), jnp.int32),
            pltpu.VMEM((rows_per_w,), jnp.int32),
            pltpu.VMEM((rows_per_w, pair_w), jnp.float32),
            pltpu.VMEM((b_per_w, N_EMB), jnp.float32),
            pltpu.SemaphoreType.DMA,
        ],
    )
    def embed_sum(tok_hbm, table_hbm, out_hbm, idx_v, half_v, rows_v, acc_v, sem):
        wid = lax.axis_index("s") * nc + lax.axis_index("c")
        base = wid * rows_per_w
        pltpu.sync_copy(tok_hbm.at[pl.ds(base, rows_per_w)], idx_v)
        for g in range(rows_per_w // lanes):
            half_v[pl.ds(g * lanes, lanes)] = (
                idx_v[pl.ds(g * lanes, lanes)] >> 1
            )
        copies = []
        for j in range(n_chunks):
            copies.append(
                pltpu.async_copy(
                    table_hbm.at[half_v.at[pl.ds(j * chunk, chunk)]],
                    rows_v.at[pl.ds(j * chunk, chunk)],
                    sem,
                )
            )
        for c in copies:
            c.wait()

        def body(r, carry):
            accs = [jnp.zeros((lanes,), jnp.float32) for _ in range(N_EMB // lanes)]
            offs_a = (idx_v[pl.ds(r * CTX, lanes)] & 1) * N_EMB
            offs_b = (idx_v[pl.ds(r * CTX + CTX - lanes, lanes)] & 1) * N_EMB
            for j in range(CTX):
                trow = r * CTX + j
                off = offs_a[j] if j < lanes else offs_b[j - (CTX - lanes)]
                for c in range(N_EMB // lanes):
                    accs[c] = accs[c] + rows_v[trow, pl.ds(off + c * lanes, lanes)]
            for c in range(N_EMB // lanes):
                acc_v[r, pl.ds(c * lanes, lanes)] = accs[c]
            return carry

        lax.fori_loop(0, b_per_w, body, 0)
        pltpu.sync_copy(acc_v, out_hbm.at[pl.ds(wid * b_per_w, b_per_w)])

    return embed_sum


MB = 32


NBUF = 2
NSPLIT = 8


def _mm_body(emb_ref, wt_ref, b_ref, out_hbm, stage_ref, sems):
    i = pl.program_id(0)
    nsteps = pl.num_programs(0)
    slot = lax.rem(i, NBUF)
    rows_per_dma = MB // NSPLIT

    @pl.when(i >= NBUF)
    def _():
        prev = i - NBUF
        for k in range(NSPLIT):
            pltpu.make_async_copy(
                stage_ref.at[slot, pl.ds(k * rows_per_dma, rows_per_dma)],
                out_hbm.at[pl.ds(prev * MB + k * rows_per_dma, rows_per_dma)],
                sems.at[slot, k],
            ).wait()

    stage_ref[slot] = (
        lax.dot_general(
            emb_ref[...].astype(jnp.bfloat16),
            wt_ref[...],
            (((1,), (0,)), ((), ())),
            preferred_element_type=jnp.float32,
        )
        + b_ref[...]
    )
    for k in range(NSPLIT):
        pltpu.make_async_copy(
            stage_ref.at[slot, pl.ds(k * rows_per_dma, rows_per_dma)],
            out_hbm.at[pl.ds(i * MB + k * rows_per_dma, rows_per_dma)],
            sems.at[slot, k],
        ).start()

    @pl.when(i == nsteps - 1)
    def _():
        for b in range(NBUF):
            step = i - b
            s = lax.rem(step, NBUF)
            for k in range(NSPLIT):
                pltpu.make_async_copy(
                    stage_ref.at[s, pl.ds(k * rows_per_dma, rows_per_dma)],
                    out_hbm.at[pl.ds(step * MB + k * rows_per_dma, rows_per_dma)],
                    sems.at[s, k],
                ).wait()


def _projection(emb, wt_bf16, bias2d):
    grid = (BATCH // MB,)
    return pl.pallas_call(
        _mm_body,
        grid=grid,
        in_specs=[
            pl.BlockSpec((MB, N_EMB), lambda i: (i, 0)),
            pl.BlockSpec((N_EMB, N_VOCAB), lambda i: (0, 0)),
            pl.BlockSpec((1, N_VOCAB), lambda i: (0, 0)),
        ],
        out_specs=pl.BlockSpec(memory_space=pl.ANY),
        out_shape=jax.ShapeDtypeStruct((BATCH, N_VOCAB), jnp.float32),
        scratch_shapes=[
            pltpu.VMEM((NBUF, MB, N_VOCAB), jnp.float32),
            pltpu.SemaphoreType.DMA((NBUF, NSPLIT)),
        ],
    )(emb, wt_bf16, bias2d)


@jax.jit
def kernel(input_token, embedding_table, fc_weight, fc_bias):
    tok = input_token.reshape(-1).astype(jnp.int32)
    table_view = embedding_table.reshape(N_VOCAB // 2, 2 * N_EMB)
    emb = _make_embed_sum()(tok, table_view)
    wt_bf16 = fc_weight.T.astype(jnp.bfloat16)
    return _projection(emb, wt_bf16, fc_bias.reshape(1, -1))
